# Initial kernel scaffold; baseline (speedup 1.0000x reference)
import jax
import jax.numpy as jnp
from jax.experimental import pallas as pl


def kernel(x, edge_index, edge_type, graph_ids, head_ids, tail_ids, rel_labels, basis, comp, Wself, bias, rel_table, fcW, fcb):
    raise NotImplementedError("write your pallas kernel here")


# speedup vs baseline: 2.9078x; 2.9078x over previous
import functools

import jax
import jax.numpy as jnp
from jax import lax
from jax.experimental import pallas as pl
from jax.experimental.pallas import tpu as pltpu
from jax.experimental.pallas import tpu_sc as plsc

N = 10000
E = 160000
D = 128
R = 200
RD = 32
NLAYERS = 3
NB = 2
B = 100

NC = 2
NS = 16
NW = NC * NS

CHUNK = 128
E_PER_W = 5120
EPAD = NW * E_PER_W
N_CHUNKS = E_PER_W // CHUNK

NPAD = 10112
ROWS_PER_TILE = NPAD // NS
RPAD = 208
BPAD = 128
IPAD = 104
NPAD2 = 12288
NODES_PER_W = NPAD2 // NW
POOL_CHUNKS = NODES_PER_W // CHUNK

F32 = jnp.float32
I32 = jnp.int32

_MESH = plsc.VectorSubcoreMesh(core_axis_name="c", subcore_axis_name="s")
_SC_PARAMS = pltpu.CompilerParams(use_tc_tiling_on_sc=False)


def _zero_rows(zsrc, dst_sh, base, total, zrows):
    off = 0
    while off < total:
        sz = min(zrows, total - off)
        pltpu.sync_copy(zsrc.at[pl.ds(0, sz)], dst_sh.at[pl.ds(base + off, sz)])
        off += sz


def _edge_body(compute_deg, *refs):
    if compute_deg:
        (hb0, hb1, src_h, dst_h, et_h, ct_h,
         aggp, degp,
         agg_sh, deg_sh, buf0, buf1, srcv, dstv, etv,
         cbuf, ones, sem0, sem1, sem2) = refs
    else:
        (hb0, hb1, src_h, dst_h, et_h, ct_h,
         aggp,
         agg_sh, deg_sh, buf0, buf1, srcv, dstv, etv,
         cbuf, ones, sem0, sem1, sem2) = refs
        degp = None

    cid = lax.axis_index("c")
    sid = lax.axis_index("s")
    wid = sid * NC + cid

    z16 = jnp.zeros((16,), F32)
    o16 = jnp.ones((16,), F32)

    def fill_zero(i, _):
        for j in range(D // 16):
            buf0[i, pl.ds(j * 16, 16)] = z16
        ones[i, pl.ds(0, 16)] = z16
        return _

    lax.fori_loop(0, CHUNK, fill_zero, None)

    base = sid * ROWS_PER_TILE
    _zero_rows(buf0, agg_sh, base, ROWS_PER_TILE, CHUNK)
    if compute_deg:
        _zero_rows(ones, deg_sh, base, ROWS_PER_TILE, CHUNK)

        def fill_one(i, _):
            ones[i, pl.ds(0, 16)] = o16
            return _

        lax.fori_loop(0, CHUNK, fill_one, None)
    plsc.subcore_barrier()

    def chunk_body(k, _):
        ebase = pl.multiple_of(wid * E_PER_W + k * CHUNK, CHUNK)
        pltpu.sync_copy(src_h.at[pl.ds(ebase, CHUNK)], srcv)
        pltpu.sync_copy(dst_h.at[pl.ds(ebase, CHUNK)], dstv)
        pltpu.sync_copy(et_h.at[pl.ds(ebase, CHUNK)], etv)
        cp0 = pltpu.async_copy(hb0.at[srcv], buf0, sem0)
        cp1 = pltpu.async_copy(hb1.at[srcv], buf1, sem1)
        cpc = pltpu.async_copy(ct_h.at[etv], cbuf, sem2)
        cp0.wait()
        cp1.wait()
        cpc.wait()

        def erow(i, _):
            a = cbuf[i, pl.ds(0, 16)]
            b = cbuf[i, pl.ds(16, 16)]
            for j in range(D // 16):
                s = pl.ds(j * 16, 16)
                buf0[i, s] = buf0[i, s] * a + buf1[i, s] * b
            return _

        lax.fori_loop(0, CHUNK, erow, None)
        pltpu.sync_copy(buf0, agg_sh.at[dstv], add=True)
        if compute_deg:
            pltpu.sync_copy(ones, deg_sh.at[dstv], add=True)
        return _

    lax.fori_loop(0, N_CHUNKS, chunk_body, None)
    plsc.subcore_barrier()

    rows = pl.ds(base, ROWS_PER_TILE)
    pltpu.sync_copy(agg_sh.at[rows], aggp.at[cid].at[rows])
    if compute_deg:
        pltpu.sync_copy(deg_sh.at[rows], degp.at[cid].at[rows])


def _make_edge_kernel(compute_deg):
    outs = [jax.ShapeDtypeStruct((NC, NPAD, D), F32)]
    if compute_deg:
        outs.append(jax.ShapeDtypeStruct((NC, NPAD, 16), F32))
    return pl.kernel(
        functools.partial(_edge_body, compute_deg),
        out_type=tuple(outs),
        mesh=_MESH,
        compiler_params=_SC_PARAMS,
        scratch_types=[
            pltpu.VMEM_SHARED((NPAD, D), F32),
            pltpu.VMEM_SHARED((NPAD, 16), F32),
            pltpu.VMEM((CHUNK, D), F32),
            pltpu.VMEM((CHUNK, D), F32),
            pltpu.VMEM((CHUNK,), I32),
            pltpu.VMEM((CHUNK,), I32),
            pltpu.VMEM((CHUNK,), I32),
            pltpu.VMEM((CHUNK, 32), F32),
            pltpu.VMEM((CHUNK, 16), F32),
            pltpu.SemaphoreType.DMA,
            pltpu.SemaphoreType.DMA,
            pltpu.SemaphoreType.DMA,
        ],
    )


ROWS_BLK = 1000
GRID_N = N // ROWS_BLK


def _transform_body(h_ref, m_ref, hb0_ref, hb1_ref, hw_ref):
    h = h_ref[...]
    hb0_ref[...] = jnp.dot(h, m_ref[0], preferred_element_type=F32)
    hb1_ref[...] = jnp.dot(h, m_ref[1], preferred_element_type=F32)
    hw_ref[...] = jnp.dot(h, m_ref[2], preferred_element_type=F32)


def _transform(h, m):
    return pl.pallas_call(
        _transform_body,
        grid=(GRID_N,),
        in_specs=[
            pl.BlockSpec((ROWS_BLK, D), lambda i: (i, 0)),
            pl.BlockSpec((3, D, D), lambda i: (0, 0, 0)),
        ],
        out_specs=[pl.BlockSpec((ROWS_BLK, D), lambda i: (i, 0))] * 3,
        out_shape=[jax.ShapeDtypeStruct((N, D), F32)] * 3,
    )(h, m)


def _combine_body(has_next, agg_ref, deg_ref, hw_ref, b_ref, m_ref,
                  h_ref, *next_refs):
    agg = agg_ref[0] + agg_ref[1]
    deg = jnp.maximum(deg_ref[0, :, 0:1] + deg_ref[1, :, 0:1], 1.0)
    h = jnp.maximum(agg / deg + hw_ref[...] + b_ref[...], 0.0)
    h_ref[...] = h
    if has_next:
        nhb0, nhb1, nhw = next_refs
        nhb0[...] = jnp.dot(h, m_ref[0], preferred_element_type=F32)
        nhb1[...] = jnp.dot(h, m_ref[1], preferred_element_type=F32)
        nhw[...] = jnp.dot(h, m_ref[2], preferred_element_type=F32)


def _combine(aggp, degp, hw, bias_l, m_next):
    has_next = m_next is not None
    n_out = 4 if has_next else 1
    if m_next is None:
        m_next = jnp.zeros((3, D, D), F32)
    return pl.pallas_call(
        functools.partial(_combine_body, has_next),
        grid=(GRID_N,),
        in_specs=[
            pl.BlockSpec((NC, ROWS_BLK, D), lambda i: (0, i, 0)),
            pl.BlockSpec((NC, ROWS_BLK, 16), lambda i: (0, i, 0)),
            pl.BlockSpec((ROWS_BLK, D), lambda i: (i, 0)),
            pl.BlockSpec((1, D), lambda i: (0, 0)),
            pl.BlockSpec((3, D, D), lambda i: (0, 0, 0)),
        ],
        out_specs=[pl.BlockSpec((ROWS_BLK, D), lambda i: (i, 0))] * n_out,
        out_shape=[jax.ShapeDtypeStruct((N, D), F32)] * n_out,
    )(aggp, degp, hw, bias_l, m_next)


def _pool_body(h1, h2, h3, gid_h, head_h, tail_h, rel_h, relt_h,
               gs1p, gs2p, gs3p, cntp, hd1, hd2, hd3, tl1, tl2, tl3, rele,
               gs1, gs2, gs3, cnt_sh, buf, gidv, idxv, brel, ones, sem0):
    cid = lax.axis_index("c")
    sid = lax.axis_index("s")
    wid = sid * NC + cid

    z16 = jnp.zeros((16,), F32)
    o16 = jnp.ones((16,), F32)

    def fill_zero(i, _):
        for j in range(D // 16):
            buf[i, pl.ds(j * 16, 16)] = z16
        ones[i, pl.ds(0, 16)] = z16
        return _

    lax.fori_loop(0, CHUNK, fill_zero, None)

    base = sid * (BPAD // NS)
    for sh in (gs1, gs2, gs3):
        pltpu.sync_copy(buf.at[pl.ds(0, BPAD // NS)], sh.at[pl.ds(base, BPAD // NS)])
    pltpu.sync_copy(ones.at[pl.ds(0, BPAD // NS)], cnt_sh.at[pl.ds(base, BPAD // NS)])

    def fill_one(i, _):
        ones[i, pl.ds(0, 16)] = o16
        return _

    lax.fori_loop(0, CHUNK, fill_one, None)
    plsc.subcore_barrier()

    nbase = wid * NODES_PER_W
    for c in range(POOL_CHUNKS):
        b0 = pl.multiple_of(nbase + c * CHUNK, CHUNK)
        pltpu.sync_copy(gid_h.at[pl.ds(b0, CHUNK)], gidv)
        for (h_t, g_sh) in ((h1, gs1), (h2, gs2), (h3, gs3)):
            pltpu.sync_copy(h_t.at[pl.ds(b0, CHUNK)], buf)
            pltpu.sync_copy(buf, g_sh.at[gidv], add=True)
        pltpu.sync_copy(ones, cnt_sh.at[gidv], add=True)

    tasks = ((head_h, h1, hd1), (head_h, h2, hd2), (head_h, h3, hd3),
             (tail_h, h1, tl1), (tail_h, h2, tl2), (tail_h, h3, tl3))
    for t, (ids_h, tab, out) in enumerate(tasks):
        @pl.when(wid == t)
        def _():
            pltpu.sync_copy(ids_h, idxv)
            pltpu.async_copy(tab.at[idxv], buf.at[pl.ds(0, IPAD)], sem0).wait()
            pltpu.sync_copy(buf.at[pl.ds(0, IPAD)], out)

    @pl.when(wid == 6)
    def _():
        pltpu.sync_copy(rel_h, idxv)
        pltpu.async_copy(relt_h.at[idxv], brel, sem0).wait()
        pltpu.sync_copy(brel, rele)

    plsc.subcore_barrier()
    rows = pl.ds(base, BPAD // NS)
    pltpu.sync_copy(gs1.at[rows], gs1p.at[cid].at[rows])
    pltpu.sync_copy(gs2.at[rows], gs2p.at[cid].at[rows])
    pltpu.sync_copy(gs3.at[rows], gs3p.at[cid].at[rows])
    pltpu.sync_copy(cnt_sh.at[rows], cntp.at[cid].at[rows])


_pool_kernel = pl.kernel(
    _pool_body,
    out_type=(
        jax.ShapeDtypeStruct((NC, BPAD, D), F32),
        jax.ShapeDtypeStruct((NC, BPAD, D), F32),
        jax.ShapeDtypeStruct((NC, BPAD, D), F32),
        jax.ShapeDtypeStruct((NC, BPAD, 16), F32),
        jax.ShapeDtypeStruct((IPAD, D), F32),
        jax.ShapeDtypeStruct((IPAD, D), F32),
        jax.ShapeDtypeStruct((IPAD, D), F32),
        jax.ShapeDtypeStruct((IPAD, D), F32),
        jax.ShapeDtypeStruct((IPAD, D), F32),
        jax.ShapeDtypeStruct((IPAD, D), F32),
        jax.ShapeDtypeStruct((IPAD, RD), F32),
    ),
    mesh=_MESH,
    compiler_params=_SC_PARAMS,
    scratch_types=[
        pltpu.VMEM_SHARED((BPAD, D), F32),
        pltpu.VMEM_SHARED((BPAD, D), F32),
        pltpu.VMEM_SHARED((BPAD, D), F32),
        pltpu.VMEM_SHARED((BPAD, 16), F32),
        pltpu.VMEM((CHUNK, D), F32),
        pltpu.VMEM((CHUNK,), I32),
        pltpu.VMEM((IPAD,), I32),
        pltpu.VMEM((IPAD, RD), F32),
        pltpu.VMEM((CHUNK, 16), F32),
        pltpu.SemaphoreType.DMA,
    ],
)


def _final_body(gs1_ref, gs2_ref, gs3_ref, cnt_ref,
                hd1_ref, hd2_ref, hd3_ref, tl1_ref, tl2_ref, tl3_ref,
                rel_ref, w_ref, b_ref, out_ref):
    cnt = jnp.maximum(cnt_ref[0, :, 0:1] + cnt_ref[1, :, 0:1], 1.0)
    acc = jnp.zeros((BPAD, 1), F32)
    for i, gref in enumerate((gs1_ref, gs2_ref, gs3_ref)):
        g = (gref[0] + gref[1]) / cnt
        acc = acc + jnp.dot(g, w_ref[pl.ds(i * D, D)],
                            preferred_element_type=F32)
    acc = acc[0:IPAD]
    for i, href in enumerate((hd1_ref, hd2_ref, hd3_ref)):
        acc = acc + jnp.dot(href[...], w_ref[pl.ds(384 + i * D, D)],
                            preferred_element_type=F32)
    for i, tref in enumerate((tl1_ref, tl2_ref, tl3_ref)):
        acc = acc + jnp.dot(tref[...], w_ref[pl.ds(768 + i * D, D)],
                            preferred_element_type=F32)
    acc = acc + jnp.dot(rel_ref[...], w_ref[pl.ds(1152, RD)],
                        preferred_element_type=F32)
    out_ref[...] = jnp.broadcast_to(acc + b_ref[0, 0], (IPAD, D))


def _final(gs1p, gs2p, gs3p, cntp, hd1, hd2, hd3, tl1, tl2, tl3, rele, fcW, fcb):
    return pl.pallas_call(
        _final_body,
        out_shape=jax.ShapeDtypeStruct((IPAD, D), F32),
    )(gs1p, gs2p, gs3p, cntp, hd1, hd2, hd3, tl1, tl2, tl3, rele, fcW, fcb)


def kernel(x, edge_index, edge_type, graph_ids, head_ids, tail_ids, rel_labels,
           basis, comp, Wself, bias, rel_table, fcW, fcb):
    src = jnp.concatenate([edge_index[0], jnp.zeros((EPAD - E,), I32)])
    dst = jnp.concatenate([edge_index[1], jnp.full((EPAD - E,), N, I32)])
    et = jnp.concatenate([edge_type, jnp.zeros((EPAD - E,), I32)])

    ms = [jnp.concatenate([basis[l], Wself[l][None]], axis=0)
          for l in range(NLAYERS)]
    cts = [jnp.pad(
        jnp.concatenate([jnp.broadcast_to(comp[l, :, 0:1], (R, 16)),
                         jnp.broadcast_to(comp[l, :, 1:2], (R, 16))], axis=1),
        ((0, RPAD - R), (0, 0))) for l in range(NLAYERS)]

    edge_deg = _make_edge_kernel(True)
    edge_plain = _make_edge_kernel(False)

    hb0, hb1, hw = _transform(x, ms[0])
    aggp, degp = edge_deg(hb0, hb1, src, dst, et, cts[0])
    h1, hb0, hb1, hw = _combine(aggp, degp, hw, bias[0][None], ms[1])
    (aggp,) = edge_plain(hb0, hb1, src, dst, et, cts[1])
    h2, hb0, hb1, hw = _combine(aggp, degp, hw, bias[1][None], ms[2])
    (aggp,) = edge_plain(hb0, hb1, src, dst, et, cts[2])
    (h3,) = _combine(aggp, degp, hw, bias[2][None], None)

    pad_n = NPAD2 - N
    h1p = jnp.pad(h1, ((0, pad_n), (0, 0)))
    h2p = jnp.pad(h2, ((0, pad_n), (0, 0)))
    h3p = jnp.pad(h3, ((0, pad_n), (0, 0)))
    gidp = jnp.concatenate([graph_ids, jnp.full((pad_n,), BPAD - 1, I32)])
    headp = jnp.pad(head_ids, (0, IPAD - B))
    tailp = jnp.pad(tail_ids, (0, IPAD - B))
    relp = jnp.pad(rel_labels, (0, IPAD - B))

    outs = _pool_kernel(h1p, h2p, h3p, gidp, headp, tailp, relp, rel_table)
    final = _final(*outs, fcW, fcb[None])
    return final[:B, 0:1]


# speedup vs baseline: 3.6900x; 1.2690x over previous
import functools

import jax
import jax.numpy as jnp
from jax import lax
from jax.experimental import pallas as pl
from jax.experimental.pallas import tpu as pltpu
from jax.experimental.pallas import tpu_sc as plsc

N = 10000
E = 160000
D = 128
R = 200
RD = 32
NLAYERS = 3
NB = 2
B = 100

NC = 2
NS = 16
NW = NC * NS

CHUNK = 128
ECHUNK = 64
EPAD = 163840
E_PER_T = EPAD // NS
N_CHUNKS = E_PER_T // ECHUNK
DH = D // 2
WDEG = DH + 16

NPAD = 10112
ROWS_PER_TILE = NPAD // NS
RPAD = 208
BPAD = 128
IPAD = 104
NPAD2 = 12288
NODES_PER_W = NPAD2 // NW
POOL_CHUNKS = NODES_PER_W // CHUNK

F32 = jnp.float32
I32 = jnp.int32

_MESH = plsc.VectorSubcoreMesh(core_axis_name="c", subcore_axis_name="s")
_SC_PARAMS = pltpu.CompilerParams(use_tc_tiling_on_sc=False)


def _zero_rows(zsrc, dst_sh, base, total, zrows):
    off = 0
    while off < total:
        sz = min(zrows, total - off)
        pltpu.sync_copy(zsrc.at[pl.ds(0, sz)], dst_sh.at[pl.ds(base + off, sz)])
        off += sz


def _edge_body(compute_deg, *refs):
    w = WDEG if compute_deg else DH
    (hbT, src_h, dst_h, et_h, ct_h, aggp,
     agg_sh, srcv, dstv, etv,
     gb0, gb1, cb0, cb1, mb0, mb1,
     sg0, sg1, ss0, ss1) = refs

    cid = lax.axis_index("c")
    sid = lax.axis_index("s")

    gbufs = (gb0, gb1)
    cbufs = (cb0, cb1)
    mbufs = (mb0, mb1)
    sgs = (sg0, sg1)
    sss = (ss0, ss1)

    z16 = jnp.zeros((16,), F32)
    o16 = jnp.ones((16,), F32)

    @plsc.parallel_loop(0, ECHUNK, unroll=2)
    def _(i):
        for j in range(w // 16):
            mb0[i, pl.ds(j * 16, 16)] = z16
            mb1[i, pl.ds(j * 16, 16)] = z16

    base = sid * ROWS_PER_TILE
    _zero_rows(mb0, agg_sh, base, ROWS_PER_TILE, ECHUNK)
    if compute_deg:
        @plsc.parallel_loop(0, ECHUNK, unroll=2)
        def _(i):
            mb0[i, pl.ds(DH, 16)] = o16
            mb1[i, pl.ds(DH, 16)] = o16
    plsc.subcore_barrier()

    r0 = pl.multiple_of(sid * N_CHUNKS, N_CHUNKS)
    pltpu.sync_copy(src_h.at[cid].at[pl.ds(r0, N_CHUNKS)], srcv)
    pltpu.sync_copy(dst_h.at[pl.ds(r0, N_CHUNKS)], dstv)
    pltpu.sync_copy(et_h.at[pl.ds(r0, N_CHUNKS)], etv)

    def issue_g(c, b):
        pltpu.async_copy(hbT.at[srcv.at[c]], gbufs[b], sgs[b])
        pltpu.async_copy(ct_h.at[etv.at[c]], cbufs[b], sgs[b])

    def wait_g(b):
        pltpu.make_async_copy(hbT.at[pl.ds(0, ECHUNK)], gbufs[b], sgs[b]).wait()
        pltpu.make_async_copy(ct_h.at[pl.ds(0, ECHUNK)], cbufs[b], sgs[b]).wait()

    def issue_s(c, b):
        pltpu.async_copy(mbufs[b], agg_sh.at[dstv.at[c]], sss[b], add=True)

    def wait_s(b):
        pltpu.make_async_copy(aggp.at[0].at[pl.ds(0, ECHUNK)],
                              mbufs[b], sss[b]).wait()

    def compute(b):
        gb = gbufs[b]
        cb = cbufs[b]
        mb = mbufs[b]

        @plsc.parallel_loop(0, ECHUNK, unroll=2)
        def _(i):
            a = cb[i, pl.ds(0, 16)]
            bb = cb[i, pl.ds(16, 16)]
            for j in range(DH // 16):
                s = pl.ds(j * 16, 16)
                mb[i, s] = gb[i, s] * a + gb[i, pl.ds(DH + j * 16, 16)] * bb

    issue_g(0, 0)
    issue_g(1, 1)

    def pair_body(k2, _):
        for b in range(2):
            c = 2 * k2 + b
            wait_g(b)

            @pl.when(k2 > 0)
            def _():
                wait_s(b)

            compute(b)
            issue_g(jnp.minimum(c + 2, N_CHUNKS - 1), b)
            issue_s(c, b)
        return _

    lax.fori_loop(0, N_CHUNKS // 2, pair_body, None)
    for b in range(2):
        wait_g(b)
        wait_s(b)
    plsc.subcore_barrier()

    rows = pl.ds(base, ROWS_PER_TILE)
    pltpu.sync_copy(agg_sh.at[rows], aggp.at[cid].at[rows])


def _make_edge_kernel(compute_deg):
    w = WDEG if compute_deg else DH
    return pl.kernel(
        functools.partial(_edge_body, compute_deg),
        out_type=(jax.ShapeDtypeStruct((NC, NPAD, w), F32),),
        mesh=_MESH,
        compiler_params=_SC_PARAMS,
        scratch_types=[
            pltpu.VMEM_SHARED((NPAD, w), F32),
            pltpu.VMEM((N_CHUNKS, ECHUNK), I32),
            pltpu.VMEM((N_CHUNKS, ECHUNK), I32),
            pltpu.VMEM((N_CHUNKS, ECHUNK), I32),
            pltpu.VMEM((ECHUNK, D), F32),
            pltpu.VMEM((ECHUNK, D), F32),
            pltpu.VMEM((ECHUNK, 32), F32),
            pltpu.VMEM((ECHUNK, 32), F32),
            pltpu.VMEM((ECHUNK, w), F32),
            pltpu.VMEM((ECHUNK, w), F32),
            pltpu.SemaphoreType.DMA,
            pltpu.SemaphoreType.DMA,
            pltpu.SemaphoreType.DMA,
            pltpu.SemaphoreType.DMA,
        ],
    )


ROWS_BLK = 1000
GRID_N = N // ROWS_BLK


def _write_tables(hbT_ref, hw_ref, h, m_ref):
    y0 = jnp.dot(h, m_ref[0], preferred_element_type=F32)
    y1 = jnp.dot(h, m_ref[1], preferred_element_type=F32)
    hbT_ref[0] = jnp.concatenate([y0[:, :DH], y1[:, :DH]], axis=1)
    hbT_ref[1] = jnp.concatenate([y0[:, DH:], y1[:, DH:]], axis=1)
    hw_ref[...] = jnp.dot(h, m_ref[2], preferred_element_type=F32)


def _transform_body(h_ref, m_ref, hbT_ref, hw_ref):
    _write_tables(hbT_ref, hw_ref, h_ref[...], m_ref)


def _transform(h, m):
    return pl.pallas_call(
        _transform_body,
        grid=(GRID_N,),
        in_specs=[
            pl.BlockSpec((ROWS_BLK, D), lambda i: (i, 0)),
            pl.BlockSpec((3, D, D), lambda i: (0, 0, 0)),
        ],
        out_specs=[pl.BlockSpec((2, ROWS_BLK, D), lambda i: (0, i, 0)),
                   pl.BlockSpec((ROWS_BLK, D), lambda i: (i, 0))],
        out_shape=[jax.ShapeDtypeStruct((2, N, D), F32),
                   jax.ShapeDtypeStruct((N, D), F32)],
    )(h, m)


def _combine_body(has_next, agg_ref, deg_ref, hw_ref, b_ref, m_ref,
                  h_ref, *next_refs):
    agg = jnp.concatenate([agg_ref[0, :, 0:DH], agg_ref[1, :, 0:DH]], axis=1)
    deg = jnp.maximum(deg_ref[0, :, DH:DH + 1], 1.0)
    h = jnp.maximum(agg / deg + hw_ref[...] + b_ref[...], 0.0)
    h_ref[...] = h
    if has_next:
        nhbT, nhw = next_refs
        _write_tables(nhbT, nhw, h, m_ref)


def _combine(aggp, degsrc, hw, bias_l, m_next):
    has_next = m_next is not None
    w = aggp.shape[-1]
    if m_next is None:
        m_next = jnp.zeros((3, D, D), F32)
    out_specs = [pl.BlockSpec((ROWS_BLK, D), lambda i: (i, 0))]
    out_shape = [jax.ShapeDtypeStruct((N, D), F32)]
    if has_next:
        out_specs += [pl.BlockSpec((2, ROWS_BLK, D), lambda i: (0, i, 0)),
                      pl.BlockSpec((ROWS_BLK, D), lambda i: (i, 0))]
        out_shape += [jax.ShapeDtypeStruct((2, N, D), F32),
                      jax.ShapeDtypeStruct((N, D), F32)]
    return pl.pallas_call(
        functools.partial(_combine_body, has_next),
        grid=(GRID_N,),
        in_specs=[
            pl.BlockSpec((NC, ROWS_BLK, w), lambda i: (0, i, 0)),
            pl.BlockSpec((1, ROWS_BLK, WDEG), lambda i: (0, i, 0)),
            pl.BlockSpec((ROWS_BLK, D), lambda i: (i, 0)),
            pl.BlockSpec((1, D), lambda i: (0, 0)),
            pl.BlockSpec((3, D, D), lambda i: (0, 0, 0)),
        ],
        out_specs=out_specs,
        out_shape=out_shape,
    )(aggp, degsrc, hw, bias_l, m_next)


def _pool_body(h1, h2, h3, gid_h, head_h, tail_h, rel_h, relt_h,
               gs1p, gs2p, gs3p, cntp, hd1, hd2, hd3, tl1, tl2, tl3, rele,
               gs1, gs2, gs3, cnt_sh, buf, gidv, idxv, brel, ones, sem0):
    cid = lax.axis_index("c")
    sid = lax.axis_index("s")
    wid = sid * NC + cid

    z16 = jnp.zeros((16,), F32)
    o16 = jnp.ones((16,), F32)

    def fill_zero(i, _):
        for j in range(D // 16):
            buf[i, pl.ds(j * 16, 16)] = z16
        ones[i, pl.ds(0, 16)] = z16
        return _

    lax.fori_loop(0, CHUNK, fill_zero, None)

    base = sid * (BPAD // NS)
    for sh in (gs1, gs2, gs3):
        pltpu.sync_copy(buf.at[pl.ds(0, BPAD // NS)], sh.at[pl.ds(base, BPAD // NS)])
    pltpu.sync_copy(ones.at[pl.ds(0, BPAD // NS)], cnt_sh.at[pl.ds(base, BPAD // NS)])

    def fill_one(i, _):
        ones[i, pl.ds(0, 16)] = o16
        return _

    lax.fori_loop(0, CHUNK, fill_one, None)
    plsc.subcore_barrier()

    nbase = wid * NODES_PER_W
    for c in range(POOL_CHUNKS):
        b0 = pl.multiple_of(nbase + c * CHUNK, CHUNK)
        pltpu.sync_copy(gid_h.at[pl.ds(b0, CHUNK)], gidv)
        for (h_t, g_sh) in ((h1, gs1), (h2, gs2), (h3, gs3)):
            pltpu.sync_copy(h_t.at[pl.ds(b0, CHUNK)], buf)
            pltpu.sync_copy(buf, g_sh.at[gidv], add=True)
        pltpu.sync_copy(ones, cnt_sh.at[gidv], add=True)

    tasks = ((head_h, h1, hd1), (head_h, h2, hd2), (head_h, h3, hd3),
             (tail_h, h1, tl1), (tail_h, h2, tl2), (tail_h, h3, tl3))
    for t, (ids_h, tab, out) in enumerate(tasks):
        @pl.when(wid == t)
        def _():
            pltpu.sync_copy(ids_h, idxv)
            pltpu.async_copy(tab.at[idxv], buf.at[pl.ds(0, IPAD)], sem0).wait()
            pltpu.sync_copy(buf.at[pl.ds(0, IPAD)], out)

    @pl.when(wid == 6)
    def _():
        pltpu.sync_copy(rel_h, idxv)
        pltpu.async_copy(relt_h.at[idxv], brel, sem0).wait()
        pltpu.sync_copy(brel, rele)

    plsc.subcore_barrier()
    rows = pl.ds(base, BPAD // NS)
    pltpu.sync_copy(gs1.at[rows], gs1p.at[cid].at[rows])
    pltpu.sync_copy(gs2.at[rows], gs2p.at[cid].at[rows])
    pltpu.sync_copy(gs3.at[rows], gs3p.at[cid].at[rows])
    pltpu.sync_copy(cnt_sh.at[rows], cntp.at[cid].at[rows])


_pool_kernel = pl.kernel(
    _pool_body,
    out_type=(
        jax.ShapeDtypeStruct((NC, BPAD, D), F32),
        jax.ShapeDtypeStruct((NC, BPAD, D), F32),
        jax.ShapeDtypeStruct((NC, BPAD, D), F32),
        jax.ShapeDtypeStruct((NC, BPAD, 16), F32),
        jax.ShapeDtypeStruct((IPAD, D), F32),
        jax.ShapeDtypeStruct((IPAD, D), F32),
        jax.ShapeDtypeStruct((IPAD, D), F32),
        jax.ShapeDtypeStruct((IPAD, D), F32),
        jax.ShapeDtypeStruct((IPAD, D), F32),
        jax.ShapeDtypeStruct((IPAD, D), F32),
        jax.ShapeDtypeStruct((IPAD, RD), F32),
    ),
    mesh=_MESH,
    compiler_params=_SC_PARAMS,
    scratch_types=[
        pltpu.VMEM_SHARED((BPAD, D), F32),
        pltpu.VMEM_SHARED((BPAD, D), F32),
        pltpu.VMEM_SHARED((BPAD, D), F32),
        pltpu.VMEM_SHARED((BPAD, 16), F32),
        pltpu.VMEM((CHUNK, D), F32),
        pltpu.VMEM((CHUNK,), I32),
        pltpu.VMEM((IPAD,), I32),
        pltpu.VMEM((IPAD, RD), F32),
        pltpu.VMEM((CHUNK, 16), F32),
        pltpu.SemaphoreType.DMA,
    ],
)


def _final_body(gs1_ref, gs2_ref, gs3_ref, cnt_ref,
                hd1_ref, hd2_ref, hd3_ref, tl1_ref, tl2_ref, tl3_ref,
                rel_ref, w_ref, b_ref, out_ref):
    cnt = jnp.maximum(cnt_ref[0, :, 0:1] + cnt_ref[1, :, 0:1], 1.0)
    acc = jnp.zeros((BPAD, 1), F32)
    for i, gref in enumerate((gs1_ref, gs2_ref, gs3_ref)):
        g = (gref[0] + gref[1]) / cnt
        acc = acc + jnp.dot(g, w_ref[pl.ds(i * D, D)],
                            preferred_element_type=F32)
    acc = acc[0:IPAD]
    for i, href in enumerate((hd1_ref, hd2_ref, hd3_ref)):
        acc = acc + jnp.dot(href[...], w_ref[pl.ds(384 + i * D, D)],
                            preferred_element_type=F32)
    for i, tref in enumerate((tl1_ref, tl2_ref, tl3_ref)):
        acc = acc + jnp.dot(tref[...], w_ref[pl.ds(768 + i * D, D)],
                            preferred_element_type=F32)
    acc = acc + jnp.dot(rel_ref[...], w_ref[pl.ds(1152, RD)],
                        preferred_element_type=F32)
    out_ref[...] = jnp.broadcast_to(acc + b_ref[0, 0], (IPAD, D))


def _final(gs1p, gs2p, gs3p, cntp, hd1, hd2, hd3, tl1, tl2, tl3, rele, fcW, fcb):
    return pl.pallas_call(
        _final_body,
        out_shape=jax.ShapeDtypeStruct((IPAD, D), F32),
    )(gs1p, gs2p, gs3p, cntp, hd1, hd2, hd3, tl1, tl2, tl3, rele, fcW, fcb)


def kernel(x, edge_index, edge_type, graph_ids, head_ids, tail_ids, rel_labels,
           basis, comp, Wself, bias, rel_table, fcW, fcb):
    src1 = jnp.concatenate([edge_index[0], jnp.zeros((EPAD - E,), I32)])
    src = jnp.stack([src1, src1 + N]).reshape(NC, EPAD // ECHUNK, ECHUNK)
    dst = jnp.concatenate([edge_index[1], jnp.full((EPAD - E,), N, I32)]
                          ).reshape(EPAD // ECHUNK, ECHUNK)
    et = jnp.concatenate([edge_type, jnp.zeros((EPAD - E,), I32)]
                         ).reshape(EPAD // ECHUNK, ECHUNK)

    ms = [jnp.concatenate([basis[l], Wself[l][None]], axis=0)
          for l in range(NLAYERS)]
    cts = [jnp.pad(
        jnp.concatenate([jnp.broadcast_to(comp[l, :, 0:1], (R, 16)),
                         jnp.broadcast_to(comp[l, :, 1:2], (R, 16))], axis=1),
        ((0, RPAD - R), (0, 0))) for l in range(NLAYERS)]

    edge_deg = _make_edge_kernel(True)
    edge_plain = _make_edge_kernel(False)

    hbT2, hw = _transform(x, ms[0])
    (agg1,) = edge_deg(hbT2.reshape(2 * N, D), src, dst, et, cts[0])
    h1, hbT2, hw = _combine(agg1, agg1, hw, bias[0][None], ms[1])
    (aggp,) = edge_plain(hbT2.reshape(2 * N, D), src, dst, et, cts[1])
    h2, hbT2, hw = _combine(aggp, agg1, hw, bias[1][None], ms[2])
    (aggp,) = edge_plain(hbT2.reshape(2 * N, D), src, dst, et, cts[2])
    (h3,) = _combine(aggp, agg1, hw, bias[2][None], None)

    pad_n = NPAD2 - N
    h1p = jnp.pad(h1, ((0, pad_n), (0, 0)))
    h2p = jnp.pad(h2, ((0, pad_n), (0, 0)))
    h3p = jnp.pad(h3, ((0, pad_n), (0, 0)))
    gidp = jnp.concatenate([graph_ids, jnp.full((pad_n,), BPAD - 1, I32)])
    headp = jnp.pad(head_ids, (0, IPAD - B))
    tailp = jnp.pad(tail_ids, (0, IPAD - B))
    relp = jnp.pad(rel_labels, (0, IPAD - B))

    outs = _pool_kernel(h1p, h2p, h3p, gidp, headp, tailp, relp, rel_table)
    final = _final(*outs, fcW, fcb[None])
    return final[:B, 0:1]
